# baseline (speedup 1.0000x reference)
import functools

import jax
import jax.numpy as jnp
from jax import lax
from jax.experimental import pallas as pl
from jax.experimental.pallas import tpu as pltpu
from jax.experimental.pallas import tpu_sc as plsc

N_ROWS = 10000
D = 128
E = 320000
CHUNK = 128
NS = 16
NC = 2
N_CHUNKS = 2560
E_PAD = N_CHUNKS * CHUNK
TILE_CHUNKS = N_CHUNKS // NS
SUB = 16
N_SUB = TILE_CHUNKS // SUB
NB = 2
STRIPE = 624
ZROWS = 32
ROW_BLOCK = 1000


def _sc_body(table_hbm, dst_hbm, src_hbm, vals_hbm, out_hbm,
             didx_v, sidx_v, vals_v, rows_v, didxc_v, zbuf_v, acc_sh,
             sg0, sg1, ss0, ss1):
    cid = lax.axis_index("c")
    sid = lax.axis_index("s")
    row0 = sid * STRIPE
    sgs = (sg0, sg1)
    sss = (ss0, ss1)

    zeros16 = jnp.zeros((16,), jnp.float32)

    def _zrow(r, _):
        for j in range(D // 16):
            zbuf_v[r, pl.ds(16 * j, 16)] = zeros16
        return 0

    lax.fori_loop(0, ZROWS, _zrow, 0)

    for b in range(2):
        for k in range(STRIPE // ZROWS):
            pltpu.sync_copy(zbuf_v, acc_sh.at[pl.ds(row0 + ZROWS * k, ZROWS)])
        pltpu.sync_copy(zbuf_v.at[pl.ds(0, STRIPE % ZROWS)],
                        acc_sh.at[pl.ds(row0 + STRIPE // ZROWS * ZROWS,
                                        STRIPE % ZROWS)])

        @pl.when(sid == NS - 1)
        def _():
            pltpu.sync_copy(zbuf_v.at[pl.ds(0, 16)],
                            acc_sh.at[pl.ds(NS * STRIPE, 16)])

        plsc.subcore_barrier()

        seg = 2 * b + cid

        def _subspan(s, _):
            crow0 = TILE_CHUNKS * sid + SUB * s
            pltpu.sync_copy(dst_hbm.at[seg, pl.ds(crow0, SUB)], didx_v)
            pltpu.sync_copy(src_hbm.at[seg, pl.ds(crow0, SUB)], sidx_v)
            pltpu.sync_copy(vals_hbm.at[seg, pl.ds(crow0, SUB)], vals_v)

            pltpu.async_copy(table_hbm.at[sidx_v.at[0]],
                             rows_v.at[0], sgs[0])

            def _scale(kl, slot):
                def _grp(g, _):
                    vv = vals_v[kl, pl.ds(16 * g, 16)]
                    for e in range(16):
                        splat = jnp.full((16,), vv[e], jnp.float32)
                        r = 16 * g + e
                        for j in range(D // 16):
                            sl = rows_v[slot, r, pl.ds(16 * j, 16)]
                            rows_v[slot, r, pl.ds(16 * j, 16)] = sl * splat
                    return 0

                lax.fori_loop(0, CHUNK // 16, _grp, 0)

            def _step(gi, _):
                for bslot in range(NB):
                    kl = NB * gi + bslot
                    other = (bslot + 1) % NB
                    pltpu.make_async_copy(table_hbm.at[sidx_v.at[kl]],
                                          rows_v.at[bslot], sgs[bslot]).wait()
                    _scale(kl, bslot)
                    for j in range(CHUNK // 16):
                        didxc_v[bslot, pl.ds(16 * j, 16)] = (
                            didx_v[kl, pl.ds(16 * j, 16)])
                    @pl.when(kl + 1 < SUB)
                    def _():
                        pltpu.async_copy(table_hbm.at[sidx_v.at[kl + 1]],
                                         rows_v.at[other], sgs[other])

                    pltpu.async_copy(rows_v.at[bslot],
                                     acc_sh.at[didxc_v.at[bslot]],
                                     sss[bslot], add=True).wait()
                return 0

            lax.fori_loop(0, SUB // NB, _step, 0)
            return 0

        lax.fori_loop(0, N_SUB, _subspan, 0)
        plsc.subcore_barrier()

        for k in range(STRIPE // ZROWS):
            r0 = row0 + ZROWS * k
            pltpu.sync_copy(acc_sh.at[pl.ds(r0, ZROWS)],
                            out_hbm.at[b, cid, pl.ds(r0, ZROWS)])

        @pl.when(sid == NS - 1)
        def _():
            pltpu.sync_copy(acc_sh.at[pl.ds(NS * STRIPE, 16)],
                            out_hbm.at[b, cid, pl.ds(NS * STRIPE, 16)])

        plsc.subcore_barrier()


@functools.partial(jax.jit)
def _sc_spmm(table_cat, dst_idx, src_idx, vals):
    mesh = plsc.VectorSubcoreMesh(core_axis_name="c", subcore_axis_name="s")
    return pl.kernel(
        _sc_body,
        out_type=jax.ShapeDtypeStruct((2, NC, N_ROWS, D), jnp.float32),
        mesh=mesh,
        scratch_types=[
            pltpu.VMEM((SUB, CHUNK), jnp.int32),
            pltpu.VMEM((SUB, CHUNK), jnp.int32),
            pltpu.VMEM((SUB, CHUNK), jnp.float32),
            pltpu.VMEM((NB, CHUNK, D), jnp.float32),
            pltpu.VMEM((NB, CHUNK), jnp.int32),
            pltpu.VMEM((ZROWS, D), jnp.float32),
            pltpu.VMEM_SHARED((N_ROWS, D), jnp.float32),
            pltpu.SemaphoreType.DMA,
            pltpu.SemaphoreType.DMA,
            pltpu.SemaphoreType.DMA,
            pltpu.SemaphoreType.DMA,
        ],
    )(table_cat, dst_idx, src_idx, vals)


def _dense_body(agg_ref, uw_ref, iw_ref, ue_ref, ie_ref, ues_ref, ies_ref):
    uw = uw_ref[...]
    iw = iw_ref[...]
    z0 = lax.dot(agg_ref[0, 0], uw, preferred_element_type=jnp.float32)
    z1 = lax.dot(agg_ref[1, 0], uw, preferred_element_type=jnp.float32)
    ues_ref[0] = jax.nn.sigmoid(z0)
    ues_ref[1] = jax.nn.sigmoid(z1)
    ue_ref[...] = jax.nn.sigmoid(0.5 * (z0 + z1))
    y0 = lax.dot(agg_ref[0, 1], iw, preferred_element_type=jnp.float32)
    y1 = lax.dot(agg_ref[1, 1], iw, preferred_element_type=jnp.float32)
    ies_ref[0] = jax.nn.sigmoid(y0)
    ies_ref[1] = jax.nn.sigmoid(y1)
    ie_ref[...] = jax.nn.sigmoid(0.5 * (y0 + y1))


@functools.partial(jax.jit)
def _dense_stage(aggs, u_w, i_w):
    grid = (N_ROWS // ROW_BLOCK,)
    agg_spec = pl.BlockSpec((2, NC, ROW_BLOCK, D), lambda i: (0, 0, i, 0))
    w_spec = pl.BlockSpec((D, D), lambda i: (0, 0))
    out_spec2 = pl.BlockSpec((ROW_BLOCK, D), lambda i: (i, 0))
    out_spec3 = pl.BlockSpec((2, ROW_BLOCK, D), lambda i: (0, i, 0))
    return pl.pallas_call(
        _dense_body,
        grid=grid,
        in_specs=[agg_spec, w_spec, w_spec],
        out_specs=[out_spec2, out_spec2, out_spec3, out_spec3],
        out_shape=[
            jax.ShapeDtypeStruct((N_ROWS, D), jnp.float32),
            jax.ShapeDtypeStruct((N_ROWS, D), jnp.float32),
            jax.ShapeDtypeStruct((2, N_ROWS, D), jnp.float32),
            jax.ShapeDtypeStruct((2, N_ROWS, D), jnp.float32),
        ],
    )(aggs, u_w, i_w)


def kernel(user_embedding, item_embedding, edge_index_b0, vals_u2i_b0,
           vals_i2u_b0, edge_index_b1, vals_u2i_b1, vals_i2u_b1, u_w, i_w):
    table_cat = jnp.concatenate([item_embedding, user_embedding], axis=0)
    u0 = edge_index_b0[0].astype(jnp.int32)
    i0 = edge_index_b0[1].astype(jnp.int32)
    u1 = edge_index_b1[0].astype(jnp.int32)
    i1 = edge_index_b1[1].astype(jnp.int32)

    def _seg(x):
        return jnp.pad(x, (0, E_PAD - E)).reshape(N_CHUNKS, CHUNK)

    dst_idx = jnp.stack([_seg(u0), _seg(i0), _seg(u1), _seg(i1)])
    src_idx = jnp.stack([_seg(i0), _seg(u0 + N_ROWS),
                         _seg(i1), _seg(u1 + N_ROWS)])
    vals = jnp.stack([_seg(vals_u2i_b0), _seg(vals_i2u_b0),
                      _seg(vals_u2i_b1), _seg(vals_i2u_b1)])
    aggs = _sc_spmm(table_cat, dst_idx, src_idx, vals)
    user_emb, item_emb, user_embeddings, item_embeddings = _dense_stage(
        aggs, u_w, i_w)
    return (user_emb, item_emb, user_embeddings, item_embeddings)


# speedup vs baseline: 1.0376x; 1.0376x over previous
import functools

import jax
import jax.numpy as jnp
from jax import lax
from jax.experimental import pallas as pl
from jax.experimental.pallas import tpu as pltpu
from jax.experimental.pallas import tpu_sc as plsc

N_ROWS = 10000
D = 128
E = 320000
CHUNK = 128
NS = 16
NC = 2
N_CHUNKS = 2560
E_PAD = N_CHUNKS * CHUNK
TILE_CHUNKS = N_CHUNKS // NS
GRP = 8
N_GRP = TILE_CHUNKS // GRP
NB = 2
STRIPE = 624
ZROWS = 32
ROW_BLOCK = 1000


def _sc_body(table_hbm, dst_hbm, src_hbm, vals_hbm, out_hbm,
             didx_v, sidx_v, vals_v, rows_v, zbuf_v, acc_sh,
             sg0, sg1, ss0, ss1):
    cid = lax.axis_index("c")
    sid = lax.axis_index("s")
    row0 = sid * STRIPE
    sgs = (sg0, sg1)
    sss = (ss0, ss1)

    zeros16 = jnp.zeros((16,), jnp.float32)

    def _zrow(r, _):
        for j in range(D // 16):
            zbuf_v[r, pl.ds(16 * j, 16)] = zeros16
        return 0

    lax.fori_loop(0, ZROWS, _zrow, 0)

    for b in range(2):
        for k in range(STRIPE // ZROWS):
            pltpu.sync_copy(zbuf_v, acc_sh.at[pl.ds(row0 + ZROWS * k, ZROWS)])
        pltpu.sync_copy(zbuf_v.at[pl.ds(0, STRIPE % ZROWS)],
                        acc_sh.at[pl.ds(row0 + STRIPE // ZROWS * ZROWS,
                                        STRIPE % ZROWS)])

        @pl.when(sid == NS - 1)
        def _():
            pltpu.sync_copy(zbuf_v.at[pl.ds(0, 16)],
                            acc_sh.at[pl.ds(NS * STRIPE, 16)])

        plsc.subcore_barrier()

        seg = 2 * b + cid

        def _group(gi, _):
            crow0 = TILE_CHUNKS * sid + GRP * gi
            pltpu.sync_copy(dst_hbm.at[seg, pl.ds(crow0, GRP)], didx_v)
            pltpu.sync_copy(src_hbm.at[seg, pl.ds(crow0, GRP)], sidx_v)
            pltpu.sync_copy(vals_hbm.at[seg, pl.ds(crow0, GRP)], vals_v)

            def _scale(j, slot):
                def _grp(g, _):
                    vv = vals_v[j, pl.ds(16 * g, 16)]
                    for e in range(16):
                        splat = jnp.full((16,), vv[e], jnp.float32)
                        r = 16 * g + e
                        for dj in range(D // 16):
                            sl = rows_v[slot, r, pl.ds(16 * dj, 16)]
                            rows_v[slot, r, pl.ds(16 * dj, 16)] = sl * splat
                    return 0

                lax.fori_loop(0, CHUNK // 16, _grp, 0)

            gath = [None, None]
            for j in range(2):
                gath[j % NB] = pltpu.async_copy(
                    table_hbm.at[sidx_v.at[j]], rows_v.at[j % NB], sgs[j % NB])
            for j in range(GRP):
                slot = j % NB
                gath[slot].wait()
                _scale(j, slot)
                pltpu.async_copy(rows_v.at[slot], acc_sh.at[didx_v.at[j]],
                                 sss[0], add=True).wait()
                if j + 2 < GRP:
                    gath[slot] = pltpu.async_copy(
                        table_hbm.at[sidx_v.at[j + 2]], rows_v.at[slot],
                        sgs[slot])
            return 0

        lax.fori_loop(0, N_GRP, _group, 0)
        plsc.subcore_barrier()

        for k in range(STRIPE // ZROWS):
            r0 = row0 + ZROWS * k
            pltpu.sync_copy(acc_sh.at[pl.ds(r0, ZROWS)],
                            out_hbm.at[b, cid, pl.ds(r0, ZROWS)])

        @pl.when(sid == NS - 1)
        def _():
            pltpu.sync_copy(acc_sh.at[pl.ds(NS * STRIPE, 16)],
                            out_hbm.at[b, cid, pl.ds(NS * STRIPE, 16)])

        plsc.subcore_barrier()


@functools.partial(jax.jit)
def _sc_spmm(table_cat, dst_idx, src_idx, vals):
    mesh = plsc.VectorSubcoreMesh(core_axis_name="c", subcore_axis_name="s")
    return pl.kernel(
        _sc_body,
        out_type=jax.ShapeDtypeStruct((2, NC, N_ROWS, D), jnp.float32),
        mesh=mesh,
        scratch_types=[
            pltpu.VMEM((GRP, CHUNK), jnp.int32),
            pltpu.VMEM((GRP, CHUNK), jnp.int32),
            pltpu.VMEM((GRP, CHUNK), jnp.float32),
            pltpu.VMEM((NB, CHUNK, D), jnp.float32),
            pltpu.VMEM((ZROWS, D), jnp.float32),
            pltpu.VMEM_SHARED((N_ROWS, D), jnp.float32),
            pltpu.SemaphoreType.DMA,
            pltpu.SemaphoreType.DMA,
            pltpu.SemaphoreType.DMA,
            pltpu.SemaphoreType.DMA,
        ],
    )(table_cat, dst_idx, src_idx, vals)


def _dense_body(agg_ref, uw_ref, iw_ref, ue_ref, ie_ref, ues_ref, ies_ref):
    uw = uw_ref[...]
    iw = iw_ref[...]
    z0 = lax.dot(agg_ref[0, 0], uw, preferred_element_type=jnp.float32)
    z1 = lax.dot(agg_ref[1, 0], uw, preferred_element_type=jnp.float32)
    ues_ref[0] = jax.nn.sigmoid(z0)
    ues_ref[1] = jax.nn.sigmoid(z1)
    ue_ref[...] = jax.nn.sigmoid(0.5 * (z0 + z1))
    y0 = lax.dot(agg_ref[0, 1], iw, preferred_element_type=jnp.float32)
    y1 = lax.dot(agg_ref[1, 1], iw, preferred_element_type=jnp.float32)
    ies_ref[0] = jax.nn.sigmoid(y0)
    ies_ref[1] = jax.nn.sigmoid(y1)
    ie_ref[...] = jax.nn.sigmoid(0.5 * (y0 + y1))


@functools.partial(jax.jit)
def _dense_stage(aggs, u_w, i_w):
    grid = (N_ROWS // ROW_BLOCK,)
    agg_spec = pl.BlockSpec((2, NC, ROW_BLOCK, D), lambda i: (0, 0, i, 0))
    w_spec = pl.BlockSpec((D, D), lambda i: (0, 0))
    out_spec2 = pl.BlockSpec((ROW_BLOCK, D), lambda i: (i, 0))
    out_spec3 = pl.BlockSpec((2, ROW_BLOCK, D), lambda i: (0, i, 0))
    return pl.pallas_call(
        _dense_body,
        grid=grid,
        in_specs=[agg_spec, w_spec, w_spec],
        out_specs=[out_spec2, out_spec2, out_spec3, out_spec3],
        out_shape=[
            jax.ShapeDtypeStruct((N_ROWS, D), jnp.float32),
            jax.ShapeDtypeStruct((N_ROWS, D), jnp.float32),
            jax.ShapeDtypeStruct((2, N_ROWS, D), jnp.float32),
            jax.ShapeDtypeStruct((2, N_ROWS, D), jnp.float32),
        ],
    )(aggs, u_w, i_w)


def kernel(user_embedding, item_embedding, edge_index_b0, vals_u2i_b0,
           vals_i2u_b0, edge_index_b1, vals_u2i_b1, vals_i2u_b1, u_w, i_w):
    table_cat = jnp.concatenate([item_embedding, user_embedding], axis=0)
    u0 = edge_index_b0[0].astype(jnp.int32)
    i0 = edge_index_b0[1].astype(jnp.int32)
    u1 = edge_index_b1[0].astype(jnp.int32)
    i1 = edge_index_b1[1].astype(jnp.int32)

    def _seg(x):
        return jnp.pad(x, (0, E_PAD - E)).reshape(N_CHUNKS, CHUNK)

    dst_idx = jnp.stack([_seg(u0), _seg(i0), _seg(u1), _seg(i1)])
    src_idx = jnp.stack([_seg(i0), _seg(u0 + N_ROWS),
                         _seg(i1), _seg(u1 + N_ROWS)])
    vals = jnp.stack([_seg(vals_u2i_b0), _seg(vals_i2u_b0),
                      _seg(vals_u2i_b1), _seg(vals_i2u_b1)])
    aggs = _sc_spmm(table_cat, dst_idx, src_idx, vals)
    user_emb, item_emb, user_embeddings, item_embeddings = _dense_stage(
        aggs, u_w, i_w)
    return (user_emb, item_emb, user_embeddings, item_embeddings)


# speedup vs baseline: 1.6719x; 1.6112x over previous
import functools

import jax
import jax.numpy as jnp
from jax import lax
from jax.experimental import pallas as pl
from jax.experimental.pallas import tpu as pltpu
from jax.experimental.pallas import tpu_sc as plsc

N_ROWS = 10000
D = 128
E = 320000
CHUNK = 128
N_CHUNKS = E // CHUNK
NS = 16
NC = 2
STRIPE = 624
ZROWS = 208
ROW_BLOCK = 1000


def _sc_body(table_hbm, dst_hbm, src_hbm, vals_hbm, out_hbm,
             didx_v, sidx_v, vals_v, rows_v, zbuf_v, acc_sh, sem, sem_i):
    cid = lax.axis_index("c")
    sid = lax.axis_index("s")
    row0 = sid * STRIPE

    zeros16 = jnp.zeros((16,), jnp.float32)

    def _zrow(r, _):
        for j in range(D // 16):
            zbuf_v[r, pl.ds(16 * j, 16)] = zeros16
        return 0

    lax.fori_loop(0, ZROWS, _zrow, 0)

    def _mul_group(g, _):
        vv = vals_v[pl.ds(16 * g, 16)]
        for e in range(16):
            splat = jnp.full((16,), vv[e], jnp.float32)
            r = 16 * g + e
            for j in range(D // 16):
                sl = rows_v[r, pl.ds(16 * j, 16)]
                rows_v[r, pl.ds(16 * j, 16)] = sl * splat
        return 0

    for b in range(2):
        for k in range(STRIPE // ZROWS):
            pltpu.sync_copy(zbuf_v, acc_sh.at[pl.ds(row0 + ZROWS * k, ZROWS)])

        @pl.when(sid == NS - 1)
        def _():
            pltpu.sync_copy(zbuf_v.at[pl.ds(0, 16)],
                            acc_sh.at[pl.ds(NS * STRIPE, 16)])

        plsc.subcore_barrier()

        eb = (2 * b + cid) * E

        def _chunk(k, _):
            c = sid + NS * k

            @pl.when(c < N_CHUNKS)
            def _():
                off = eb + c * CHUNK
                ca = pltpu.async_copy(dst_hbm.at[pl.ds(off, CHUNK)], didx_v,
                                      sem_i)
                cb = pltpu.async_copy(src_hbm.at[pl.ds(off, CHUNK)], sidx_v,
                                      sem_i)
                cc = pltpu.async_copy(vals_hbm.at[pl.ds(off, CHUNK)], vals_v,
                                      sem_i)
                ca.wait()
                cb.wait()
                cc.wait()
                pltpu.async_copy(table_hbm.at[sidx_v], rows_v, sem).wait()
                lax.fori_loop(0, CHUNK // 16, _mul_group, 0)
                pltpu.sync_copy(rows_v, acc_sh.at[didx_v], add=True)

            return 0

        lax.fori_loop(0, (N_CHUNKS + NS - 1) // NS, _chunk, 0)
        plsc.subcore_barrier()

        for k in range(STRIPE // ZROWS):
            r0 = row0 + ZROWS * k
            pltpu.sync_copy(acc_sh.at[pl.ds(r0, ZROWS)],
                            out_hbm.at[b, cid, pl.ds(r0, ZROWS)])

        @pl.when(sid == NS - 1)
        def _():
            pltpu.sync_copy(acc_sh.at[pl.ds(NS * STRIPE, 16)],
                            out_hbm.at[b, cid, pl.ds(NS * STRIPE, 16)])

        plsc.subcore_barrier()


@functools.partial(jax.jit, donate_argnums=())
def _sc_spmm(table_cat, dst_idx, src_idx, vals):
    mesh = plsc.VectorSubcoreMesh(core_axis_name="c", subcore_axis_name="s")
    return pl.kernel(
        _sc_body,
        out_type=jax.ShapeDtypeStruct((2, NC, N_ROWS, D), jnp.float32),
        mesh=mesh,
        scratch_types=[
            pltpu.VMEM((CHUNK,), jnp.int32),
            pltpu.VMEM((CHUNK,), jnp.int32),
            pltpu.VMEM((CHUNK,), jnp.float32),
            pltpu.VMEM((CHUNK, D), jnp.float32),
            pltpu.VMEM((ZROWS, D), jnp.float32),
            pltpu.VMEM_SHARED((N_ROWS, D), jnp.float32),
            pltpu.SemaphoreType.DMA,
            pltpu.SemaphoreType.DMA,
        ],
    )(table_cat, dst_idx, src_idx, vals)


def _dense_body(agg_ref, uw_ref, iw_ref, ue_ref, ie_ref, ues_ref, ies_ref):
    uw = uw_ref[...]
    iw = iw_ref[...]
    z0 = lax.dot(agg_ref[0, 0], uw, preferred_element_type=jnp.float32)
    z1 = lax.dot(agg_ref[1, 0], uw, preferred_element_type=jnp.float32)
    ues_ref[0] = jax.nn.sigmoid(z0)
    ues_ref[1] = jax.nn.sigmoid(z1)
    ue_ref[...] = jax.nn.sigmoid(0.5 * (z0 + z1))
    y0 = lax.dot(agg_ref[0, 1], iw, preferred_element_type=jnp.float32)
    y1 = lax.dot(agg_ref[1, 1], iw, preferred_element_type=jnp.float32)
    ies_ref[0] = jax.nn.sigmoid(y0)
    ies_ref[1] = jax.nn.sigmoid(y1)
    ie_ref[...] = jax.nn.sigmoid(0.5 * (y0 + y1))


@functools.partial(jax.jit)
def _dense_stage(aggs, u_w, i_w):
    grid = (N_ROWS // ROW_BLOCK,)
    agg_spec = pl.BlockSpec((2, NC, ROW_BLOCK, D), lambda i: (0, 0, i, 0))
    w_spec = pl.BlockSpec((D, D), lambda i: (0, 0))
    out_spec2 = pl.BlockSpec((ROW_BLOCK, D), lambda i: (i, 0))
    out_spec3 = pl.BlockSpec((2, ROW_BLOCK, D), lambda i: (0, i, 0))
    return pl.pallas_call(
        _dense_body,
        grid=grid,
        in_specs=[agg_spec, w_spec, w_spec],
        out_specs=[out_spec2, out_spec2, out_spec3, out_spec3],
        out_shape=[
            jax.ShapeDtypeStruct((N_ROWS, D), jnp.float32),
            jax.ShapeDtypeStruct((N_ROWS, D), jnp.float32),
            jax.ShapeDtypeStruct((2, N_ROWS, D), jnp.float32),
            jax.ShapeDtypeStruct((2, N_ROWS, D), jnp.float32),
        ],
    )(aggs, u_w, i_w)


def kernel(user_embedding, item_embedding, edge_index_b0, vals_u2i_b0,
           vals_i2u_b0, edge_index_b1, vals_u2i_b1, vals_i2u_b1, u_w, i_w):
    table_cat = jnp.concatenate([item_embedding, user_embedding], axis=0)
    u0 = edge_index_b0[0].astype(jnp.int32)
    i0 = edge_index_b0[1].astype(jnp.int32)
    u1 = edge_index_b1[0].astype(jnp.int32)
    i1 = edge_index_b1[1].astype(jnp.int32)
    dst_idx = jnp.concatenate([u0, i0, u1, i1])
    src_idx = jnp.concatenate([i0, u0 + N_ROWS, i1, u1 + N_ROWS])
    vals = jnp.concatenate([vals_u2i_b0, vals_i2u_b0,
                            vals_u2i_b1, vals_i2u_b1])
    aggs = _sc_spmm(table_cat, dst_idx, src_idx, vals)
    user_emb, item_emb, user_embeddings, item_embeddings = _dense_stage(
        aggs, u_w, i_w)
    return (user_emb, item_emb, user_embeddings, item_embeddings)
